# baseline (speedup 1.0000x reference)
import jax
import jax.numpy as jnp
from jax import lax
from jax.experimental import pallas as pl
from jax.experimental.pallas import tpu as pltpu
from jax.experimental.pallas import tpu_sc as plsc

_N = 10000
_NPAD = 10112
_NE = 320000
_D = 128
_EAW = 16
_CNT_COL = 16
_EPSBN = 1e-5

_NC = 2
_NS = 16
_NW = _NC * _NS
_CH = 128
_NCHUNK = 80
_EPW = _CH * _NCHUNK
_NE_PAD = _EPW * _NW
_RPW = _NPAD // _NS
_RCHUNKS = [(k * _CH, min(_CH, _RPW - k * _CH))
            for k in range((_RPW + _CH - 1) // _CH)]


def _zero_rows(buf, nrows):
  def zero_row(i, _):
    for l in range(_D // 16):
      buf[i, pl.ds(l * 16, 16)] = jnp.zeros((16,), jnp.float32)
    return 0
  lax.fori_loop(0, nrows, zero_row, 0)


def _init_acc(acc, rows_zero, s):
  r0 = pl.multiple_of(s * _RPW, 8)
  for off, sz in _RCHUNKS:
    pltpu.sync_copy(rows_zero.at[pl.ds(0, sz)],
                    acc.at[pl.ds(r0 + off, sz)])


def _drain_acc(acc, stage, out, c, s):
  r0 = pl.multiple_of(s * _RPW, 8)
  for off, sz in _RCHUNKS:
    pltpu.sync_copy(acc.at[pl.ds(r0 + off, sz)], stage.at[pl.ds(0, sz)])
    pltpu.sync_copy(stage.at[pl.ds(0, sz)], out.at[c, pl.ds(r0 + off, sz)])


def _sc_gather_segsum():
  mesh = plsc.VectorSubcoreMesh(
      core_axis_name="c", subcore_axis_name="s", num_cores=_NC,
      num_subcores=_NS)

  def body(tab, pk3, out_s, pkall, idx4, rows0, rows1, acc_s, sem0, sem1):
    c = lax.axis_index("c")
    s = lax.axis_index("s")
    wid = s * _NC + c
    pltpu.sync_copy(pk3.at[wid], pkall)
    _zero_rows(rows0, _CH)
    _init_acc(acc_s, rows0, s)
    plsc.subcore_barrier()

    rows = (rows0, rows1)
    sems = (sem0, sem1)

    def unpack(j, p):
      def lane(l, _):
        v = pkall[j, pl.ds(l * 16, 16)]
        idx4[p, pl.ds(l * 16, 16)] = lax.shift_right_logical(v, 14)
        idx4[2 + p, pl.ds(l * 16, 16)] = lax.bitwise_and(v, 16383)
        return 0
      lax.fori_loop(0, _CH // 16, lane, 0)

    unpack(0, 0)
    pltpu.async_copy(tab.at[idx4.at[0]], rows0, sem0)

    def pair(i, _):
      for b in (0, 1):
        j = 2 * i + b
        pltpu.make_async_copy(tab.at[idx4.at[b]], rows[b], sems[b]).wait()
        unpack(j + 1, 1 - b)
        pltpu.async_copy(tab.at[idx4.at[1 - b]], rows[1 - b], sems[1 - b])
        pltpu.sync_copy(rows[b], acc_s.at[idx4.at[2 + b]], add=True)
      return 0
    lax.fori_loop(0, _NCHUNK // 2, pair, 0)
    pltpu.make_async_copy(tab.at[idx4.at[0]], rows0, sem0).wait()
    plsc.subcore_barrier()
    _drain_acc(acc_s, rows0, out_s, c, s)

  return pl.kernel(
      body,
      out_type=[jax.ShapeDtypeStruct((_NC, _NPAD, _D), jnp.float32)],
      mesh=mesh,
      scratch_types=[
          pltpu.VMEM((_NCHUNK + 1, _CH), jnp.int32),
          pltpu.VMEM((4, _CH), jnp.int32),
          pltpu.VMEM((_CH, _D), jnp.float32),
          pltpu.VMEM((_CH, _D), jnp.float32),
          pltpu.VMEM_SHARED((_NPAD, _D), jnp.float32),
          pltpu.SemaphoreType.DMA,
          pltpu.SemaphoreType.DMA,
      ])


_EROWS = _CH * _EAW // _D


def _sc_edge_segsum():
  mesh = plsc.VectorSubcoreMesh(
      core_axis_name="c", subcore_axis_name="s", num_cores=_NC,
      num_subcores=_NS)

  def body(ea2, dst3, out_e, dstall, eav0, eav1, rowsv, acc_e,
           sem0, sem1):
    c = lax.axis_index("c")
    s = lax.axis_index("s")
    wid = s * _NC + c
    base = pl.multiple_of(wid * (_EPW // 8), 8)
    pltpu.sync_copy(dst3.at[wid], dstall)
    _zero_rows(rowsv, _CH)
    _init_acc(acc_e, rowsv, s)
    plsc.subcore_barrier()

    cnt_vec = jnp.where(lax.iota(jnp.int32, 16) == 0, 1.0, 0.0)
    eavs = (eav0, eav1)
    sems = (sem0, sem1)
    pltpu.async_copy(ea2.at[pl.ds(base, _EROWS)], eav0, sem0)

    def pair(i, _):
      for b in (0, 1):
        j = 2 * i + b
        off = pl.multiple_of(base + j * _EROWS, 8)
        pltpu.make_async_copy(ea2.at[pl.ds(off, _EROWS)], eavs[b],
                              sems[b]).wait()
        pltpu.async_copy(ea2.at[pl.ds(off + _EROWS, _EROWS)], eavs[1 - b],
                         sems[1 - b])

        def expand(rr, _):
          for q in range(_D // _EAW):
            rowsv[rr * 8 + q, pl.ds(0, _EAW)] = eavs[b][rr, pl.ds(q * _EAW,
                                                                  _EAW)]
            rowsv[rr * 8 + q, pl.ds(_EAW, 16)] = cnt_vec
          return 0
        lax.fori_loop(0, _EROWS, expand, 0)
        pltpu.sync_copy(rowsv, acc_e.at[dstall.at[j]], add=True)
      return 0
    lax.fori_loop(0, _NCHUNK // 2, pair, 0)
    pltpu.make_async_copy(ea2.at[pl.ds(base + _EPW // 8, _EROWS)], eav0,
                          sem0).wait()
    plsc.subcore_barrier()
    _drain_acc(acc_e, rowsv, out_e, c, s)

  return pl.kernel(
      body,
      out_type=[jax.ShapeDtypeStruct((_NC, _NPAD, _D), jnp.float32)],
      mesh=mesh,
      scratch_types=[
          pltpu.VMEM((_NCHUNK, _CH), jnp.int32),
          pltpu.VMEM((_EROWS, _D), jnp.float32),
          pltpu.VMEM((_EROWS, _D), jnp.float32),
          pltpu.VMEM((_CH, _D), jnp.float32),
          pltpu.VMEM_SHARED((_NPAD, _D), jnp.float32),
          pltpu.SemaphoreType.DMA,
          pltpu.SemaphoreType.DMA,
      ])


def _tc_body(x_ref, s_ref, e_ref, wsT_ref, wnnT_ref, wneT_ref, bs_ref,
             g_ref, b_ref, o_ref):
  hi = jax.lax.Precision.HIGHEST
  S = s_ref[0] + s_ref[1]
  E = e_ref[0] + e_ref[1]
  counts = E[:, _CNT_COL:_CNT_COL + 1]
  denom = 1.0 / jnp.maximum(counts, 1.0)
  aggr = (jnp.dot(S, wnnT_ref[...], precision=hi)
          + jnp.dot(E, wneT_ref[...], precision=hi)) * denom
  xs = jnp.dot(x_ref[...], wsT_ref[...], precision=hi) + bs_ref[...]
  h = jnp.maximum(xs + aggr, 0.0)
  o_ref[...] = jnp.maximum(h * g_ref[...] + b_ref[...], 0.0)


def _tc_layer(xp, Sp, Ep, wsT, wnnT, wneT, bs, g, b):
  br = 2528
  grid = (_NPAD // br,)
  return pl.pallas_call(
      _tc_body,
      grid=grid,
      in_specs=[
          pl.BlockSpec((br, _D), lambda i: (i, 0)),
          pl.BlockSpec((_NC, br, _D), lambda i: (0, i, 0)),
          pl.BlockSpec((_NC, br, _D), lambda i: (0, i, 0)),
          pl.BlockSpec((_D, _D), lambda i: (0, 0)),
          pl.BlockSpec((_D, _D), lambda i: (0, 0)),
          pl.BlockSpec((_D, _D), lambda i: (0, 0)),
          pl.BlockSpec((1, _D), lambda i: (0, 0)),
          pl.BlockSpec((1, _D), lambda i: (0, 0)),
          pl.BlockSpec((1, _D), lambda i: (0, 0)),
      ],
      out_specs=pl.BlockSpec((br, _D), lambda i: (i, 0)),
      out_shape=jax.ShapeDtypeStruct((_NPAD, _D), jnp.float32),
  )(xp, Sp, Ep, wsT, wnnT, wneT, bs, g, b)


def _prep_neighbor_weights(Wn, bn_):
  wnnT = Wn.T[:_D]
  wneT = jnp.zeros((_D, _D), jnp.float32)
  wneT = wneT.at[:16].set(Wn.T[_D:_D + 16]).at[_CNT_COL].set(bn_)
  return wnnT, wneT


def kernel(x, edge_index, edge_attr, W1n, b1n, W1s, b1s, bn1_g, bn1_b,
           W2n, b2n, W2s, b2s, bn2_g, bn2_b):
  f32 = jnp.float32
  src = edge_index[0].astype(jnp.int32)
  dst = edge_index[1].astype(jnp.int32)
  padn = _NE_PAD - _NE
  srcp = jnp.concatenate([src, jnp.zeros((padn,), jnp.int32)])
  junk = _N + (jnp.arange(padn, dtype=jnp.int32) % (_NPAD - _N))
  dstp = jnp.concatenate([dst, junk])
  pk3 = jnp.pad((srcp * 16384 + dstp).reshape(_NW, _NCHUNK, _CH),
                ((0, 0), (0, 1), (0, 0)))
  dst3 = dstp.reshape(_NW, _NCHUNK, _CH)
  ea2 = jnp.pad(edge_attr.astype(f32),
                ((0, padn + _CH), (0, 0))).reshape(-1, _D)
  xp = jnp.zeros((_NPAD, _D), f32).at[:_N].set(x.astype(f32))

  sc_gather = _sc_gather_segsum()
  (Ep,) = _sc_edge_segsum()(ea2, dst3)
  (S1p,) = sc_gather(xp, pk3)

  bnscale = 1.0 / jnp.sqrt(1.0 + _EPSBN)
  w1nnT, w1neT = _prep_neighbor_weights(W1n, b1n)
  h = _tc_layer(xp, S1p, Ep, W1s.T, w1nnT, w1neT, b1s[None],
                (bn1_g * bnscale)[None], bn1_b[None])

  (S2p,) = sc_gather(h, pk3)
  w2nnT, w2neT = _prep_neighbor_weights(W2n, b2n)
  out = _tc_layer(h, S2p, Ep, W2s.T, w2nnT, w2neT, b2s[None],
                  (bn2_g * bnscale)[None], bn2_b[None])
  return out[:_N]
